# baseline (speedup 1.0000x reference)
import jax
import jax.numpy as jnp
from jax import lax
from jax.experimental import pallas as pl
from jax.experimental.pallas import tpu as pltpu
from jax.experimental.pallas import tpu_sc as plsc

BATCH = 4096
SEQ = 200
EMBD = 64

NC = 2
NS = 16
NW = NC * NS
B_PER_W = BATCH // NW
NBUF = 2
B0 = 499968
NROWS1 = 1000000 - B0


def _sc_body(sh0_hbm, sh1_hbm, idx_hbm, cnt_hbm, out_hbm,
             idx_v, pid_v, buf, out_v, cnt_v, rowb_v, *sems):
    wid = lax.axis_index("s") * NC + lax.axis_index("c")

    pltpu.sync_copy(idx_hbm.at[pl.ds(wid * B_PER_W, B_PER_W)], idx_v)
    pltpu.sync_copy(cnt_hbm.at[pl.ds(wid * B_PER_W, B_PER_W)], cnt_v)
    pltpu.sync_copy(sh1_hbm.at[pl.ds(0, 1)], rowb_v)

    def issue(s, k):
        for m in range(13):
            lo = min(m * 16, SEQ - 16)
            v = idx_v[s, pl.ds(lo, 16)]
            big = v >= B0
            pid_v[k, 0, pl.ds(lo, 16)] = jnp.where(big, 0, v)
            pid_v[k, 1, pl.ds(lo, 16)] = jnp.where(big, v - B0, 0)
        pltpu.async_copy(sh0_hbm.at[pid_v.at[k, 0, pl.ds(0, 128)]],
                         buf.at[k, pl.ds(0, 128)], sems[k])
        pltpu.async_copy(sh0_hbm.at[pid_v.at[k, 0, pl.ds(128, 72)]],
                         buf.at[k, pl.ds(128, 72)], sems[k])
        pltpu.async_copy(sh1_hbm.at[pid_v.at[k, 1, pl.ds(0, 128)]],
                         buf.at[k, pl.ds(200, 128)], sems[k])
        pltpu.async_copy(sh1_hbm.at[pid_v.at[k, 1, pl.ds(128, 72)]],
                         buf.at[k, pl.ds(328, 72)], sems[k])

    def drain(k):
        pltpu.make_async_copy(
            sh0_hbm.at[pl.ds(0, 2 * SEQ)], buf.at[k], sems[k]
        ).wait()

    def accum(s, k):
        def blk(i, acc):
            a0, a1, a2, a3 = acc
            for jj in range(8):
                j = i * 8 + jj
                a0 += buf[k, j, pl.ds(0, 16)]
                a1 += buf[k, j, pl.ds(16, 16)]
                a2 += buf[k, j, pl.ds(32, 16)]
                a3 += buf[k, j, pl.ds(48, 16)]
            return (a0, a1, a2, a3)

        z = jnp.zeros((16,), jnp.float32)
        a0, a1, a2, a3 = lax.fori_loop(0, 2 * SEQ // 8, blk, (z, z, z, z))
        c = cnt_v[s, pl.ds(0, 16)]
        out_v[s, pl.ds(0, 16)] = a0 - c * rowb_v[0, pl.ds(0, 16)]
        out_v[s, pl.ds(16, 16)] = a1 - c * rowb_v[0, pl.ds(16, 16)]
        out_v[s, pl.ds(32, 16)] = a2 - c * rowb_v[0, pl.ds(32, 16)]
        out_v[s, pl.ds(48, 16)] = a3 - c * rowb_v[0, pl.ds(48, 16)]

    for k in range(NBUF):
        issue(k, k)

    def step(t, _):
        for k in range(NBUF):
            s = t * NBUF + k
            drain(k)
            accum(s, k)
            nxt = s + NBUF

            @pl.when(nxt < B_PER_W)
            def _():
                issue(nxt, k)

        return 0

    lax.fori_loop(0, B_PER_W // NBUF, step, 0)
    pltpu.sync_copy(out_v, out_hbm.at[pl.ds(wid * B_PER_W, B_PER_W)])


@jax.jit
def _pooled_lookup(indices, table):
    sh0 = table[:B0]
    sh1 = table[B0:1000000]
    cnt0 = jnp.sum((indices < B0).astype(jnp.float32), axis=1)
    cnt0b = jnp.broadcast_to(cnt0[:, None], (BATCH, 16))
    mesh = plsc.VectorSubcoreMesh(core_axis_name="c", subcore_axis_name="s")
    return pl.kernel(
        _sc_body,
        out_type=jax.ShapeDtypeStruct((BATCH, EMBD), jnp.float32),
        mesh=mesh,
        scratch_types=[
            pltpu.VMEM((B_PER_W, SEQ), jnp.int32),
            pltpu.VMEM((NBUF, 2, SEQ), jnp.int32),
            pltpu.VMEM((NBUF, 2 * SEQ, EMBD), jnp.float32),
            pltpu.VMEM((B_PER_W, EMBD), jnp.float32),
            pltpu.VMEM((B_PER_W, 16), jnp.float32),
            pltpu.VMEM((1, EMBD), jnp.float32),
        ] + [pltpu.SemaphoreType.DMA] * NBUF,
        compiler_params=pltpu.CompilerParams(use_tc_tiling_on_sc=False),
    )(sh0, sh1, indices, cnt0b)


def kernel(indices, table):
    return _pooled_lookup(indices.astype(jnp.int32), table)


# speedup vs baseline: 12.5922x; 12.5922x over previous
import jax
import jax.numpy as jnp
from jax import lax
from jax.experimental import pallas as pl
from jax.experimental.pallas import tpu as pltpu
from jax.experimental.pallas import tpu_sc as plsc

BATCH = 4096
SEQ = 200
EMBD = 64

NC = 2
NS = 16
NW = NC * NS
B_PER_W = BATCH // NW
CHUNK = 40
N_CHUNKS = SEQ // CHUNK
CHUNKS_PER_W = B_PER_W * N_CHUNKS
NBUF = 4


def _sc_body(table_hbm, idx_hbm, out_hbm, idx_v, buf, out_v, *sems):
    wid = lax.axis_index("s") * NC + lax.axis_index("c")

    pltpu.sync_copy(idx_hbm.at[pl.ds(wid * B_PER_W, B_PER_W)], idx_v)

    def issue(s, k):
        for c in range(N_CHUNKS):
            pltpu.async_copy(
                table_hbm.at[idx_v.at[s, pl.ds(c * CHUNK, CHUNK)]],
                buf.at[k, pl.ds(c * CHUNK, CHUNK)],
                sems[k],
            )

    def drain(k):
        pltpu.make_async_copy(
            table_hbm.at[pl.ds(0, SEQ)], buf.at[k], sems[k]
        ).wait()

    def accum(s, k):
        def blk(i, acc):
            a0, a1, a2, a3 = acc
            for jj in range(8):
                j = i * 8 + jj
                a0 += buf[k, j, pl.ds(0, 16)]
                a1 += buf[k, j, pl.ds(16, 16)]
                a2 += buf[k, j, pl.ds(32, 16)]
                a3 += buf[k, j, pl.ds(48, 16)]
            return (a0, a1, a2, a3)

        z = jnp.zeros((16,), jnp.float32)
        a0, a1, a2, a3 = lax.fori_loop(0, SEQ // 8, blk, (z, z, z, z))
        out_v[s, pl.ds(0, 16)] = a0
        out_v[s, pl.ds(16, 16)] = a1
        out_v[s, pl.ds(32, 16)] = a2
        out_v[s, pl.ds(48, 16)] = a3

    for k in range(NBUF):
        issue(k, k)

    def step(t, _):
        for k in range(NBUF):
            s = t * NBUF + k
            drain(k)
            accum(s, k)
            nxt = s + NBUF

            @pl.when(nxt < B_PER_W)
            def _():
                issue(nxt, k)

        return 0

    lax.fori_loop(0, B_PER_W // NBUF, step, 0)
    pltpu.sync_copy(out_v, out_hbm.at[pl.ds(wid * B_PER_W, B_PER_W)])


@jax.jit
def _pooled_lookup(indices, table):
    mesh = plsc.VectorSubcoreMesh(core_axis_name="c", subcore_axis_name="s")
    return pl.kernel(
        _sc_body,
        out_type=jax.ShapeDtypeStruct((BATCH, EMBD), jnp.float32),
        mesh=mesh,
        scratch_types=[
            pltpu.VMEM((B_PER_W, SEQ), jnp.int32),
            pltpu.VMEM((NBUF, SEQ, EMBD), jnp.float32),
            pltpu.VMEM((B_PER_W, EMBD), jnp.float32),
        ] + [pltpu.SemaphoreType.DMA] * NBUF,
        compiler_params=pltpu.CompilerParams(use_tc_tiling_on_sc=False),
    )(table, indices)


def kernel(indices, table):
    return _pooled_lookup(indices.astype(jnp.int32), table)
